# baseline (speedup 1.0000x reference)
import functools

import jax
import jax.numpy as jnp
from jax import lax
from jax.experimental import pallas as pl
from jax.experimental.pallas import tpu as pltpu
from jax.experimental.pallas import tpu_sc as plsc

_B = 16384
_F = 39
_NCAT = 26
_NCONT = _F - _NCAT
_V = 100000
_D = 16
_EPS = 1e-5
_OUTW = _NCAT * _D + _NCONT

_NW = 32
_QS = 25088
_TAIL0 = 3 * _QS + 24704
_Q3 = _TAIL0 - 3 * _QS
_NTAIL = _V - _TAIL0
_QOFF = (0, _QS, 2 * _QS, 3 * _QS)
_QLEN = (_QS, _QS, _QS, _Q3 + _NTAIL)
_BCH = 2048
_NBCH = _B // _BCH
_UNROLL = 4


def _prelude_body(xT_ref, gamma_ref, beta_ref, contT_ref):
    xc = xT_ref[_NCAT:, :]
    mean = jnp.mean(xc, axis=1, keepdims=True)
    var = jnp.mean((xc - mean) ** 2, axis=1, keepdims=True)
    inv = lax.rsqrt(var + _EPS)
    contT_ref[...] = (xc - mean) * inv * gamma_ref[...] + beta_ref[...]


def _sc_body(xT, tablesT, tails, contT, outT, qrow_v, idx_v, outf_v, stage_v,
             sem_q, sem_o):
    wid = lax.axis_index("s") * 2 + lax.axis_index("c")
    start = wid * 13
    iota16 = lax.broadcasted_iota(jnp.int32, (16,), 0)

    def q_issue(t, q):
        f = t >> 4
        d = t & 15
        rb = q % 2
        pltpu.async_copy(
            tablesT.at[f, d, pl.ds(_QOFF[q], _QS if q < 3 else _Q3)],
            qrow_v.at[rb, pl.ds(0, _QS if q < 3 else _Q3)],
            sem_q,
        )
        if q == 3:
            pltpu.async_copy(
                tails.at[f, d, :], qrow_v.at[rb, pl.ds(_Q3, _NTAIL)], sem_q
            )

    def q_wait(q):
        rb = q % 2
        pltpu.make_async_copy(
            tablesT.at[0, 0, pl.ds(0, _QS if q < 3 else _Q3)],
            qrow_v.at[rb, pl.ds(0, _QS if q < 3 else _Q3)],
            sem_q,
        ).wait()
        if q == 3:
            pltpu.make_async_copy(
                tails.at[0, 0, :], qrow_v.at[rb, pl.ds(_Q3, _NTAIL)], sem_q
            ).wait()

    def out_wait(ob):
        pltpu.make_async_copy(outT.at[0, :], outf_v.at[ob], sem_o).wait()

    @pl.when(wid < _NCONT)
    def _():
        def cchunk(c, carry):
            pltpu.sync_copy(contT.at[wid, pl.ds(c * _BCH, _BCH)], stage_v)
            pltpu.sync_copy(stage_v, outT.at[_NCAT * _D + wid, pl.ds(c * _BCH, _BCH)])
            return carry

        lax.fori_loop(0, _NBCH, cchunk, 0)

    q_issue(start, 0)

    def task_body(k, prev_f):
        t = start + k
        f = t >> 4
        ob = k & 1

        @pl.when(f != prev_f)
        def _():
            def conv_chunk(c, carry2):
                b0 = c * _BCH
                pltpu.sync_copy(xT.at[f, pl.ds(b0, _BCH)], stage_v)

                def conv_body(g, carry3):
                    for j in range(_UNROLL):
                        o = (g * _UNROLL + j) * 16
                        idx_v[pl.ds(b0 + o, 16)] = stage_v[pl.ds(o, 16)].astype(
                            jnp.int32
                        )
                    return carry3

                lax.fori_loop(0, _BCH // 16 // _UNROLL, conv_body, 0)
                return carry2

            lax.fori_loop(0, _NBCH, conv_chunk, 0)

        @pl.when(k >= 2)
        def _():
            out_wait(ob)

        obs = jnp.broadcast_to(ob, (16,))
        for q in range(4):
            q_wait(q)
            if q < 3:
                q_issue(t, q + 1)
            else:

                @pl.when(k < 12)
                def _():
                    q_issue(t + 1, 0)

            rbs = jnp.broadcast_to(jnp.int32(q % 2), (16,))
            qoff = _QOFF[q]
            qlen = jnp.uint32(_QLEN[q])
            qmax = jnp.uint32(_QLEN[q] - 1)

            def g_body(g, carry3):
                for j in range(_UNROLL):
                    o = (g * _UNROLL + j) * 16
                    local = (idx_v[pl.ds(o, 16)] - qoff).astype(jnp.uint32)
                    m = local < qlen
                    safe = jnp.minimum(local, qmax).astype(jnp.int32)
                    vals = plsc.load_gather(qrow_v, [rbs, safe])
                    plsc.store_scatter(outf_v, [obs, o + iota16], vals, mask=m)
                return carry3

            lax.fori_loop(0, _B // 16 // _UNROLL, g_body, 0)

        pltpu.async_copy(outf_v.at[ob], outT.at[t, :], sem_o)
        return f

    lax.fori_loop(0, 13, task_body, jnp.int32(-1))
    out_wait(0)
    out_wait(1)


@jax.jit
def kernel(x, tables, gamma, beta):
    xT = x.T
    tablesT = jnp.transpose(tables, (0, 2, 1))
    tails = jnp.transpose(tables[:, _TAIL0:, :], (0, 2, 1))

    contT = pl.pallas_call(
        _prelude_body,
        out_shape=jax.ShapeDtypeStruct((_NCONT, _B), jnp.float32),
    )(xT, gamma.reshape(_NCONT, 1), beta.reshape(_NCONT, 1))

    sc_call = pl.kernel(
        _sc_body,
        out_type=jax.ShapeDtypeStruct((_OUTW, _B), jnp.float32),
        mesh=plsc.VectorSubcoreMesh(core_axis_name="c", subcore_axis_name="s"),
        scratch_types=[
            pltpu.VMEM((2, _QS), jnp.float32),
            pltpu.VMEM((_B,), jnp.int32),
            pltpu.VMEM((2, _B), jnp.float32),
            pltpu.VMEM((_BCH,), jnp.float32),
            pltpu.SemaphoreType.DMA,
            pltpu.SemaphoreType.DMA,
        ],
        compiler_params=pltpu.CompilerParams(
            use_tc_tiling_on_sc=True, needs_layout_passes=False
        ),
    )
    outT = sc_call(xT, tablesT, tails, contT)
    return outT.T


# speedup vs baseline: 3.5956x; 3.5956x over previous
import functools

import jax
import jax.numpy as jnp
from jax import lax
from jax.experimental import pallas as pl
from jax.experimental.pallas import tpu as pltpu
from jax.experimental.pallas import tpu_sc as plsc

_B = 16384
_F = 39
_NCAT = 26
_NCONT = _F - _NCAT
_V = 100000
_D = 16
_EPS = 1e-5
_OUTW = _NCAT * _D + _NCONT

_NW = 32
_BCH = 2048
_NBCH = _B // _BCH
_UNROLL = 8


def _prelude_body(xT_ref, gamma_ref, beta_ref, contT_ref):
    xc = xT_ref[_NCAT:, :]
    mean = jnp.mean(xc, axis=1, keepdims=True)
    var = jnp.mean((xc - mean) ** 2, axis=1, keepdims=True)
    inv = lax.rsqrt(var + _EPS)
    contT_ref[...] = (xc - mean) * inv * gamma_ref[...] + beta_ref[...]


_QS = 25088
_Q3 = 99968 - 3 * _QS
_NTAIL = 128


def _sc_body(xT, tablesT, tails, contT, outT, row_v, idx_v, outb_v, sem_r, sem_i, sem_o):
    wid = lax.axis_index("s") * 2 + lax.axis_index("c")
    start = wid * 13

    def cat_task(t, f, prev_f):
        d = t & 15
        cps = []
        for q in range(4):
            off = q * _QS
            sz = _QS if q < 3 else _Q3
            cps.append(
                pltpu.async_copy(
                    tablesT.at[f, d, pl.ds(off, sz)],
                    row_v.at[pl.ds(off, sz)],
                    sem_r,
                )
            )
        cps.append(
            pltpu.async_copy(
                tails.at[pl.ds(t * _NTAIL, _NTAIL)],
                row_v.at[pl.ds(_V - _NTAIL, _NTAIL)],
                sem_r,
            )
        )

        @pl.when(f != prev_f)
        def _():
            def conv_chunk(c, carry2):
                b0 = c * _BCH
                pltpu.sync_copy(xT.at[f, pl.ds(b0, _BCH)], outb_v.at[0])

                def conv_body(g, carry3):
                    for j in range(_UNROLL):
                        o = (g * _UNROLL + j) * 16
                        idx_v[pl.ds(b0 + o, 16)] = outb_v[0, pl.ds(o, 16)].astype(
                            jnp.int32
                        )
                    return carry3

                lax.fori_loop(0, _BCH // 16 // _UNROLL, conv_body, 0)
                return carry2

            lax.fori_loop(0, _NBCH, conv_chunk, 0)

        for cp in cps:
            cp.wait()
        cp_o = [None, None, None, None]
        for c in range(_NBCH):
            buf = c % 4
            if cp_o[buf] is not None:
                cp_o[buf].wait()
            b0 = c * _BCH

            def g_body(g, carry3):
                for j in range(_UNROLL):
                    o = (g * _UNROLL + j) * 16
                    idx16 = idx_v[pl.ds(b0 + o, 16)]
                    outb_v[buf, pl.ds(o, 16)] = plsc.load_gather(row_v, [idx16])
                return carry3

            lax.fori_loop(0, _BCH // 16 // _UNROLL, g_body, 0)
            cp_o[buf] = pltpu.async_copy(
                outb_v.at[buf], outT.at[t, pl.ds(b0, _BCH)], sem_o
            )
        for cp in cp_o:
            cp.wait()

    def cont_task(t):
        pltpu.sync_copy(contT.at[t - _NCAT * _D, :], row_v.at[pl.ds(0, _B)])
        pltpu.sync_copy(row_v.at[pl.ds(0, _B)], outT.at[t, :])

    def task_body(k, prev_f):
        t = start + k
        f = t >> 4
        cat_task(t, f, prev_f)
        return f

    @pl.when(wid < _NCONT)
    def _():
        cont_task(_NCAT * _D + wid)

    lax.fori_loop(0, 13, task_body, jnp.int32(-1))


@jax.jit
def kernel(x, tables, gamma, beta):
    xT = x.T
    tablesT = jnp.transpose(tables, (0, 2, 1))
    tails = jnp.transpose(tables[:, _V - 128 :, :], (0, 2, 1)).reshape(-1)

    contT = pl.pallas_call(
        _prelude_body,
        out_shape=jax.ShapeDtypeStruct((_NCONT, _B), jnp.float32),
    )(xT, gamma.reshape(_NCONT, 1), beta.reshape(_NCONT, 1))

    sc_call = pl.kernel(
        _sc_body,
        out_type=jax.ShapeDtypeStruct((_OUTW, _B), jnp.float32),
        mesh=plsc.VectorSubcoreMesh(core_axis_name="c", subcore_axis_name="s"),
        scratch_types=[
            pltpu.VMEM((_V,), jnp.float32),
            pltpu.VMEM((_B,), jnp.int32),
            pltpu.VMEM((4, _BCH), jnp.float32),
            pltpu.SemaphoreType.DMA,
            pltpu.SemaphoreType.DMA,
            pltpu.SemaphoreType.DMA,
        ],
        compiler_params=pltpu.CompilerParams(
            use_tc_tiling_on_sc=True, needs_layout_passes=False
        ),
    )
    outT = sc_call(xT, tablesT, tails, contT)
    return outT.T
